# baseline (speedup 1.0000x reference)
import dataclasses
import functools

import jax
import jax.numpy as jnp
from jax import lax
from jax.experimental import pallas as pl
from jax.experimental.pallas import tpu as pltpu
from jax.experimental.pallas import tpu_sc as plsc

_LANES = 16
_NUM_WORKERS = 32
_GATHER_CHUNK = 128


def _sc_hash_gather(ids, table, seqlen):
    n_tok = ids.shape[0]
    dim = table.shape[1]
    buckets = table.shape[0]
    chunk = n_tok // _NUM_WORKERS
    n_gather = chunk // _GATHER_CHUNK
    mesh = plsc.VectorSubcoreMesh(core_axis_name="c", subcore_axis_name="s")
    cparams = pltpu.CompilerParams(use_tc_tiling_on_sc=False)
    if "needs_layout_passes" in pltpu.CompilerParams.__dataclass_fields__:
        cparams = dataclasses.replace(cparams, needs_layout_passes=False)

    @functools.partial(
        pl.kernel,
        out_type=jax.ShapeDtypeStruct((n_tok, dim), jnp.float32),
        mesh=mesh,
        compiler_params=cparams,
        scratch_types=[
            pltpu.VMEM((_LANES + chunk,), jnp.int32),
            pltpu.VMEM((n_gather, _GATHER_CHUNK), jnp.int32),
            pltpu.VMEM((chunk, dim), jnp.float32),
            pltpu.SemaphoreType.DMA,
        ],
    )
    def gather_kernel(ids_hbm, table_hbm, out_hbm, ids_pad, h_ref, rows, sem):
        i32 = jnp.int32
        sub = lax.convert_element_type(lax.axis_index("s"), jnp.int32)
        core = lax.convert_element_type(lax.axis_index("c"), jnp.int32)
        wid = sub * i32(2) + core
        base = wid * i32(chunk)

        @pl.when(base % i32(seqlen) == i32(0))
        def _():
            ids_pad[pl.ds(0, _LANES)] = jnp.zeros((_LANES,), jnp.int32)
            pltpu.sync_copy(
                ids_hbm.at[pl.ds(base, chunk)], ids_pad.at[pl.ds(_LANES, chunk)]
            )

        @pl.when(base % i32(seqlen) != i32(0))
        def _():
            pltpu.sync_copy(
                ids_hbm.at[pl.ds(base - i32(_LANES), chunk + _LANES)], ids_pad
            )

        lane = lax.iota(jnp.int32, _LANES)

        @pl.loop(i32(0), i32(n_gather))
        def _(r):
            r = lax.convert_element_type(r, jnp.int32)
            for t in range(_GATHER_CHUNK // _LANES):
                off = r * i32(_GATHER_CHUNK) + i32(t * _LANES)
                cur = ids_pad[pl.ds(off + i32(_LANES), _LANES)]
                prev = plsc.load_gather(ids_pad, [lane + (off + i32(_LANES - 1))])
                h = (((prev * i32(92)) % i32(1000)) * i32(1000)
                     + prev * i32(821) + cur) % i32(buckets)
                h_ref[r, pl.ds(i32(t * _LANES), _LANES)] = h
            pltpu.make_async_copy(
                table_hbm.at[h_ref.at[r]],
                rows.at[pl.ds(r * i32(_GATHER_CHUNK), _GATHER_CHUNK)],
                sem,
            ).start()

        @pl.loop(i32(0), i32(n_gather))
        def _(r):
            r = lax.convert_element_type(r, jnp.int32)
            pltpu.make_async_copy(
                table_hbm.at[h_ref.at[r]],
                rows.at[pl.ds(r * i32(_GATHER_CHUNK), _GATHER_CHUNK)],
                sem,
            ).wait()

        pltpu.sync_copy(rows, out_hbm.at[pl.ds(base, chunk)])

    return gather_kernel(ids, table)


def _tc_project(emb, w):
    n_tok, dim = emb.shape
    model_dim = w.shape[0]
    blk = 512

    def body(emb_ref, w_ref, out_ref):
        out_ref[...] = lax.dot_general(
            emb_ref[...],
            w_ref[...],
            dimension_numbers=(((1,), (1,)), ((), ())),
            preferred_element_type=jnp.float32,
        )

    return pl.pallas_call(
        body,
        grid=(n_tok // blk,),
        in_specs=[
            pl.BlockSpec((blk, dim), lambda i: (i, jnp.int32(0))),
            pl.BlockSpec((model_dim, dim), lambda i: (jnp.int32(0), jnp.int32(0))),
        ],
        out_specs=pl.BlockSpec((blk, model_dim), lambda i: (i, jnp.int32(0))),
        out_shape=jax.ShapeDtypeStruct((n_tok, model_dim), jnp.float32),
    )(emb, w)


def kernel(input_ids, table, W):
    bsz, seqlen = input_ids.shape
    ids = input_ids.reshape(-1).astype(jnp.int32)
    emb = _sc_hash_gather(ids, table, seqlen)
    out = _tc_project(emb, W)
    return out.reshape(bsz, seqlen, W.shape[0])
